# baseline (speedup 1.0000x reference)
import functools

import jax
import jax.numpy as jnp
from jax.experimental import pallas as pl
from jax.experimental.pallas import tpu as pltpu


def _lsh_kernel(x_ref, w_ref, o_ref, *, L, Lb, R, H, CH):
    D = x_ref.shape[2]
    P = H // 2
    w = w_ref[0]
    parts = []
    for r in range(R):
        wr = jax.lax.slice(w, (0, r * P), (D, (r + 1) * P))
        parts += [wr, -wr]
    w2 = jnp.concatenate(parts, axis=1)
    Lc = Lb // CH

    def project(ci):
        x = x_ref[0, pl.ds(ci * Lc, Lc), :]
        n2 = jnp.sum(x * x, axis=1, keepdims=True)
        x = x * (1.0 / jnp.maximum(jnp.sqrt(n2), 1e-12))
        return jnp.dot(x, w2, preferred_element_type=jnp.float32)

    def reduce_store(ci, m):
        row = (jax.lax.broadcasted_iota(jnp.int32, (Lc, 1), 0)
               + (pl.program_id(1) * Lb + ci * Lc))
        outs = []
        for r in range(R):
            c = jax.lax.slice(m, (0, r * H), (Lc, (r + 1) * H))
            outs.append(jnp.argmax(c, axis=1, keepdims=True))
        o_ref[0, pl.ds(ci * Lc, Lc), :] = jnp.concatenate(outs, axis=1) * L + row

    m_prev = project(0)
    for ci in range(1, CH):
        m_cur = project(ci)
        reduce_store(ci - 1, m_prev)
        m_prev = m_cur
    reduce_store(CH - 1, m_prev)


def kernel(inp, rand_matrix, n_buckets):
    del n_buckets
    B, L, D = inp.shape
    R, P = rand_matrix.shape[2], rand_matrix.shape[3]
    H = 2 * P
    w = rand_matrix.reshape(B, D, R * P)
    Lb = 4096
    CH = 4
    grid = (B, L // Lb)
    return pl.pallas_call(
        functools.partial(_lsh_kernel, L=L, Lb=Lb, R=R, H=H, CH=CH),
        grid=grid,
        in_specs=[
            pl.BlockSpec((1, Lb, D), lambda b, i: (b, i, 0)),
            pl.BlockSpec((1, D, R * P), lambda b, i: (b, 0, 0)),
        ],
        out_specs=pl.BlockSpec((1, Lb, R), lambda b, i: (b, i, 0)),
        out_shape=jax.ShapeDtypeStruct((B, L, R), jnp.int32),
        compiler_params=pltpu.CompilerParams(
            dimension_semantics=("parallel", "parallel")),
    )(inp, w)


# speedup vs baseline: 1.0087x; 1.0087x over previous
import functools

import jax
import jax.numpy as jnp
from jax.experimental import pallas as pl
from jax.experimental.pallas import tpu as pltpu


def _lsh_kernel(x_ref, w_ref, o_ref, *, L, Lb, R, H, CH):
    D = x_ref.shape[2]
    P = H // 2
    parts = []
    for r in range(R):
        wr = w_ref[0, :, r, :]
        parts += [wr, -wr]
    w2 = jnp.concatenate(parts, axis=1)
    Lc = Lb // CH

    def project(ci):
        x = x_ref[0, pl.ds(ci * Lc, Lc), :]
        n2 = jnp.sum(x * x, axis=1, keepdims=True)
        x = x * (1.0 / jnp.maximum(jnp.sqrt(n2), 1e-12))
        return jnp.dot(x, w2, preferred_element_type=jnp.float32)

    def reduce_store(ci, m):
        row = (jax.lax.broadcasted_iota(jnp.int32, (Lc, 1), 0)
               + (pl.program_id(1) * Lb + ci * Lc))
        outs = []
        for r in range(R):
            c = jax.lax.slice(m, (0, r * H), (Lc, (r + 1) * H))
            outs.append(jnp.argmax(c, axis=1, keepdims=True))
        o_ref[0, pl.ds(ci * Lc, Lc), :] = jnp.concatenate(outs, axis=1) * L + row

    m_prev = project(0)
    for ci in range(1, CH):
        m_cur = project(ci)
        reduce_store(ci - 1, m_prev)
        m_prev = m_cur
    reduce_store(CH - 1, m_prev)


def kernel(inp, rand_matrix, n_buckets):
    del n_buckets
    B, L, D = inp.shape
    R, P = rand_matrix.shape[2], rand_matrix.shape[3]
    H = 2 * P
    Lb = 4096
    CH = 4
    grid = (B, L // Lb)
    return pl.pallas_call(
        functools.partial(_lsh_kernel, L=L, Lb=Lb, R=R, H=H, CH=CH),
        grid=grid,
        in_specs=[
            pl.BlockSpec((1, Lb, D), lambda b, i: (b, i, 0)),
            pl.BlockSpec((1, D, R, P), lambda b, i: (b, 0, 0, 0)),
        ],
        out_specs=pl.BlockSpec((1, Lb, R), lambda b, i: (b, i, 0)),
        out_shape=jax.ShapeDtypeStruct((B, L, R), jnp.int32),
        compiler_params=pltpu.CompilerParams(
            dimension_semantics=("parallel", "parallel")),
    )(inp, rand_matrix)


# speedup vs baseline: 1.3472x; 1.3356x over previous
import functools

import jax
import jax.numpy as jnp
from jax.experimental import pallas as pl
from jax.experimental.pallas import tpu as pltpu


def _lsh_kernel(x_ref, w_ref, o_ref, *, L, Lb, R, H, CH):
    D = x_ref.shape[1]
    P = H // 2
    parts = []
    for r in range(R):
        wr = w_ref[0, :, r, :]
        parts += [wr, -wr]
    w2 = jnp.concatenate(parts, axis=1)
    Lc = Lb // CH

    def project(ci):
        x = x_ref[0, :, pl.ds(ci * Lc, Lc)]
        n2 = jnp.sum(x * x, axis=0, keepdims=True)
        x = x * (1.0 / jnp.maximum(jnp.sqrt(n2), 1e-12))
        return jax.lax.dot_general(
            x, w2, (((0,), (0,)), ((), ())),
            preferred_element_type=jnp.float32)

    def reduce_store(ci, m):
        row = (jax.lax.broadcasted_iota(jnp.int32, (Lc, 1), 0)
               + (pl.program_id(1) * Lb + ci * Lc))
        outs = []
        for r in range(R):
            c = jax.lax.slice(m, (0, r * H), (Lc, (r + 1) * H))
            outs.append(jnp.argmax(c, axis=1, keepdims=True))
        o_ref[0, pl.ds(ci * Lc, Lc), :] = jnp.concatenate(outs, axis=1) * L + row

    m_prev = project(0)
    for ci in range(1, CH):
        m_cur = project(ci)
        reduce_store(ci - 1, m_prev)
        m_prev = m_cur
    reduce_store(CH - 1, m_prev)


def kernel(inp, rand_matrix, n_buckets):
    del n_buckets
    B, L, D = inp.shape
    R, P = rand_matrix.shape[2], rand_matrix.shape[3]
    H = 2 * P
    inp_t = jnp.transpose(inp, (0, 2, 1))
    Lb = 4096
    CH = 4
    grid = (B, L // Lb)
    return pl.pallas_call(
        functools.partial(_lsh_kernel, L=L, Lb=Lb, R=R, H=H, CH=CH),
        grid=grid,
        in_specs=[
            pl.BlockSpec((1, D, Lb), lambda b, i: (b, 0, i)),
            pl.BlockSpec((1, D, R, P), lambda b, i: (b, 0, 0, 0)),
        ],
        out_specs=pl.BlockSpec((1, Lb, R), lambda b, i: (b, i, 0)),
        out_shape=jax.ShapeDtypeStruct((B, L, R), jnp.int32),
        compiler_params=pltpu.CompilerParams(
            dimension_semantics=("parallel", "parallel")),
    )(inp_t, rand_matrix)


# speedup vs baseline: 1.5326x; 1.1376x over previous
import functools

import jax
import jax.numpy as jnp
from jax.experimental import pallas as pl
from jax.experimental.pallas import tpu as pltpu


def _lsh_kernel(x_ref, w_ref, o_ref, *, L, Lb, R, H, CH):
    D = x_ref.shape[1]
    P = H // 2
    parts = []
    for r in range(R):
        wr = w_ref[0, :, r, :]
        parts += [wr, -wr]
    w2 = jnp.concatenate(parts, axis=1)
    Lc = Lb // CH

    def project(ci):
        x = x_ref[0, :, pl.ds(ci * Lc, Lc)]
        n2 = jnp.sum(x * x, axis=0, keepdims=True)
        x = x * (1.0 / jnp.maximum(jnp.sqrt(n2), 1e-12))
        return jax.lax.dot_general(
            x, w2, (((0,), (0,)), ((), ())),
            preferred_element_type=jnp.float32)

    def reduce_store(ci, m):
        row = (jax.lax.broadcasted_iota(jnp.int32, (1, Lc), 1)
               + (pl.program_id(1) * Lb + ci * Lc))
        outs = []
        for r in range(R):
            c = jax.lax.slice(m, (0, r * H), (Lc, (r + 1) * H))
            outs.append(jnp.argmax(c, axis=1, keepdims=True))
        buckets = jnp.transpose(jnp.concatenate(outs, axis=1))
        o_ref[0, :, pl.ds(ci * Lc, Lc)] = buckets * L + row

    m_prev = project(0)
    for ci in range(1, CH):
        m_cur = project(ci)
        reduce_store(ci - 1, m_prev)
        m_prev = m_cur
    reduce_store(CH - 1, m_prev)


def kernel(inp, rand_matrix, n_buckets):
    del n_buckets
    B, L, D = inp.shape
    R, P = rand_matrix.shape[2], rand_matrix.shape[3]
    H = 2 * P
    inp_t = jnp.transpose(inp, (0, 2, 1))
    Lb = 4096
    CH = 4
    grid = (B, L // Lb)
    out = pl.pallas_call(
        functools.partial(_lsh_kernel, L=L, Lb=Lb, R=R, H=H, CH=CH),
        grid=grid,
        in_specs=[
            pl.BlockSpec((1, D, Lb), lambda b, i: (b, 0, i)),
            pl.BlockSpec((1, D, R, P), lambda b, i: (b, 0, 0, 0)),
        ],
        out_specs=pl.BlockSpec((1, R, Lb), lambda b, i: (b, 0, i)),
        out_shape=jax.ShapeDtypeStruct((B, R, L), jnp.int32),
        compiler_params=pltpu.CompilerParams(
            dimension_semantics=("parallel", "parallel")),
    )(inp_t, rand_matrix)
    return jnp.transpose(out, (0, 2, 1))


# speedup vs baseline: 1.5418x; 1.0060x over previous
import functools

import jax
import jax.numpy as jnp
from jax.experimental import pallas as pl
from jax.experimental.pallas import tpu as pltpu


def _lsh_kernel(x_ref, w_ref, o_ref, *, L, Lb, R, H, CH):
    D = x_ref.shape[1]
    P = H // 2
    parts = []
    for r in range(R):
        wr = w_ref[0, :, r, :]
        parts += [wr, -wr]
    w2 = jnp.concatenate(parts, axis=1)
    Lc = Lb // CH

    def project(ci):
        x = x_ref[0, :, pl.ds(ci * Lc, Lc)]
        n2 = jnp.sum(x * x, axis=0, keepdims=True)
        x = x * (1.0 / jnp.maximum(jnp.sqrt(n2), 1e-12))
        return jax.lax.dot_general(
            x, w2, (((0,), (0,)), ((), ())),
            preferred_element_type=jnp.float32)

    def reduce_store(ci, m):
        row = (jax.lax.broadcasted_iota(jnp.int32, (1, Lc), 1)
               + (pl.program_id(1) * Lb + ci * Lc))
        outs = []
        for r in range(R):
            c = jax.lax.slice(m, (0, r * H), (Lc, (r + 1) * H))
            outs.append(jnp.argmax(c, axis=1, keepdims=True))
        buckets = jnp.transpose(jnp.concatenate(outs, axis=1))
        o_ref[0, :, pl.ds(ci * Lc, Lc)] = buckets * L + row

    m_prev = project(0)
    for ci in range(1, CH):
        m_cur = project(ci)
        reduce_store(ci - 1, m_prev)
        m_prev = m_cur
    reduce_store(CH - 1, m_prev)


def kernel(inp, rand_matrix, n_buckets):
    del n_buckets
    B, L, D = inp.shape
    R, P = rand_matrix.shape[2], rand_matrix.shape[3]
    H = 2 * P
    inp_t = jnp.transpose(inp, (0, 2, 1))
    Lb = L
    CH = 4
    grid = (B, L // Lb)
    out = pl.pallas_call(
        functools.partial(_lsh_kernel, L=L, Lb=Lb, R=R, H=H, CH=CH),
        grid=grid,
        in_specs=[
            pl.BlockSpec((1, D, Lb), lambda b, i: (b, 0, i)),
            pl.BlockSpec((1, D, R, P), lambda b, i: (b, 0, 0, 0)),
        ],
        out_specs=pl.BlockSpec((1, R, Lb), lambda b, i: (b, 0, i)),
        out_shape=jax.ShapeDtypeStruct((B, R, L), jnp.int32),
        compiler_params=pltpu.CompilerParams(
            dimension_semantics=("parallel", "parallel")),
    )(inp_t, rand_matrix)
    return jnp.transpose(out, (0, 2, 1))


# speedup vs baseline: 1.6013x; 1.0386x over previous
import functools

import jax
import jax.numpy as jnp
from jax.experimental import pallas as pl
from jax.experimental.pallas import tpu as pltpu


def _lsh_kernel(x_ref, w_ref, o_ref, *, L, Lb, R, H, CH, Nb):
    D = x_ref.shape[1]
    P = H // 2
    Lc = Lb // CH

    w2s = {}

    def get_w2(bi):
        if bi not in w2s:
            parts = []
            for r in range(R):
                wr = w_ref[bi, :, r, :]
                parts += [wr, -wr]
            w2s[bi] = jnp.concatenate(parts, axis=1)
        return w2s[bi]

    def project(bi, ci):
        x = x_ref[bi, :, pl.ds(ci * Lc, Lc)]
        n2 = jnp.sum(x * x, axis=0, keepdims=True)
        x = x * (1.0 / jnp.maximum(jnp.sqrt(n2), 1e-12))
        return jax.lax.dot_general(
            x, get_w2(bi), (((0,), (0,)), ((), ())),
            preferred_element_type=jnp.float32)

    def reduce_store(bi, ci, m):
        row = (jax.lax.broadcasted_iota(jnp.int32, (1, Lc), 1)
               + (pl.program_id(1) * Lb + ci * Lc))
        outs = []
        for r in range(R):
            c = jax.lax.slice(m, (0, r * H), (Lc, (r + 1) * H))
            outs.append(jnp.argmax(c, axis=1, keepdims=True))
        buckets = jnp.transpose(jnp.concatenate(outs, axis=1))
        o_ref[bi, :, pl.ds(ci * Lc, Lc)] = buckets * L + row

    tasks = [(bi, ci) for bi in range(Nb) for ci in range(CH)]
    m_prev = project(*tasks[0])
    for t_prev, t_cur in zip(tasks, tasks[1:]):
        m_cur = project(*t_cur)
        reduce_store(*t_prev, m_prev)
        m_prev = m_cur
    reduce_store(*tasks[-1], m_prev)


def kernel(inp, rand_matrix, n_buckets):
    del n_buckets
    B, L, D = inp.shape
    R, P = rand_matrix.shape[2], rand_matrix.shape[3]
    H = 2 * P
    inp_t = jnp.transpose(inp, (0, 2, 1))
    Lb = L
    CH = 4
    Nb = 2
    grid = (B // Nb, L // Lb)
    out = pl.pallas_call(
        functools.partial(_lsh_kernel, L=L, Lb=Lb, R=R, H=H, CH=CH, Nb=Nb),
        grid=grid,
        in_specs=[
            pl.BlockSpec((Nb, D, Lb), lambda b, i: (b, 0, i)),
            pl.BlockSpec((Nb, D, R, P), lambda b, i: (b, 0, 0, 0)),
        ],
        out_specs=pl.BlockSpec((Nb, R, Lb), lambda b, i: (b, 0, i)),
        out_shape=jax.ShapeDtypeStruct((B, R, L), jnp.int32),
        compiler_params=pltpu.CompilerParams(
            dimension_semantics=("parallel", "parallel")),
    )(inp_t, rand_matrix)
    return jnp.transpose(out, (0, 2, 1))


# speedup vs baseline: 1.6261x; 1.0155x over previous
import functools

import jax
import jax.numpy as jnp
from jax.experimental import pallas as pl
from jax.experimental.pallas import tpu as pltpu


def _lsh_kernel(x_ref, w_ref, o_ref, *, L, Lb, R, H, CH, Nb):
    D = x_ref.shape[1]
    P = H // 2
    Lc = Lb // CH

    w2s = {}

    def get_w2(bi):
        if bi not in w2s:
            parts = []
            for r in range(R):
                wr = w_ref[bi, :, r, :]
                parts += [wr, -wr]
            w2s[bi] = jnp.concatenate(parts, axis=1)
        return w2s[bi]

    def project(bi, ci):
        x = x_ref[bi, :, pl.ds(ci * Lc, Lc)]
        n2 = jnp.sum(x * x, axis=0, keepdims=True)
        x = x * (1.0 / jnp.maximum(jnp.sqrt(n2), 1e-12))
        return jax.lax.dot_general(
            x, get_w2(bi), (((0,), (0,)), ((), ())),
            preferred_element_type=jnp.float32)

    def reduce_store(bi, ci, m):
        row = (jax.lax.broadcasted_iota(jnp.int32, (1, Lc), 1)
               + (pl.program_id(1) * Lb + ci * Lc))
        outs = []
        for r in range(R):
            c = jax.lax.slice(m, (0, r * H), (Lc, (r + 1) * H))
            outs.append(jnp.argmax(c, axis=1, keepdims=True))
        buckets = jnp.transpose(jnp.concatenate(outs, axis=1))
        o_ref[bi, :, pl.ds(ci * Lc, Lc)] = buckets * L + row

    tasks = [(bi, ci) for bi in range(Nb) for ci in range(CH)]
    m_prev = project(*tasks[0])
    for t_prev, t_cur in zip(tasks, tasks[1:]):
        m_cur = project(*t_cur)
        reduce_store(*t_prev, m_prev)
        m_prev = m_cur
    reduce_store(*tasks[-1], m_prev)


def kernel(inp, rand_matrix, n_buckets):
    del n_buckets
    B, L, D = inp.shape
    R, P = rand_matrix.shape[2], rand_matrix.shape[3]
    H = 2 * P
    inp_t = jnp.transpose(inp, (0, 2, 1))
    Lb = L
    CH = 4
    Nb = 4
    grid = (B // Nb, L // Lb)
    out = pl.pallas_call(
        functools.partial(_lsh_kernel, L=L, Lb=Lb, R=R, H=H, CH=CH, Nb=Nb),
        grid=grid,
        in_specs=[
            pl.BlockSpec((Nb, D, Lb), lambda b, i: (b, 0, i)),
            pl.BlockSpec((Nb, D, R, P), lambda b, i: (b, 0, 0, 0)),
        ],
        out_specs=pl.BlockSpec((Nb, R, Lb), lambda b, i: (b, 0, i)),
        out_shape=jax.ShapeDtypeStruct((B, R, L), jnp.int32),
        compiler_params=pltpu.CompilerParams(
            dimension_semantics=("parallel", "parallel")),
    )(inp_t, rand_matrix)
    return jnp.transpose(out, (0, 2, 1))
